# baseline (speedup 1.0000x reference)
import jax
import jax.numpy as jnp
import numpy as np
from jax.experimental import pallas as pl
from jax.experimental.pallas import tpu as pltpu

K_SEL = 2048


def _rowsum(x):
    def lanereduce(p):
        pt = p.T
        t = pt.reshape(16, 8, p.shape[0])
        u = t[0]
        for j in range(1, 16):
            u = u + t[j]
        h = u[0:4] + u[4:8]
        q = h[0:2] + h[2:4]
        return q[0:1] + q[1:2]

    a = lanereduce(x[:, 0:128] + x[:, 128:256])
    b = lanereduce(x[:, 256:384] + x[:, 384:512])
    c = lanereduce(x[:, 512:640] + x[:, 640:768])
    return ((a + b) + c).T


def _preact_body(x_ref, gamma_ref, beta_ref, w1_ref, b1_ref, out_ref):
    x = x_ref[...]
    mean = _rowsum(x) / 768.0
    cen = x - mean
    var = _rowsum(cen * cen) / 768.0
    xn = cen / jnp.sqrt(var + 1e-5) * gamma_ref[...] + beta_ref[...]
    r = jax.lax.dot_general(w1_ref[...], xn, (((0,), (1,)), ((), ())))
    out_ref[...] = r.T + b1_ref[...]


def _score2_body(h_ref, w2_ref, b2_ref, out_ref):
    s = jnp.dot(h_ref[...], w2_ref[...]) + b2_ref[...]
    out_ref[...] = (s[:, 0] + 0.0).reshape(out_ref.shape)


def _scores(features, gamma, beta, W1, b1, W2, b2, bl=1024):
    B, L, D = features.shape
    H = W1.shape[1]
    N = B * L
    feats = features.reshape(N, D)
    preact = pl.pallas_call(
        _preact_body,
        grid=(N // bl,),
        in_specs=[
            pl.BlockSpec((bl, D), lambda i: (i, 0)),
            pl.BlockSpec((D,), lambda i: (0,)),
            pl.BlockSpec((D,), lambda i: (0,)),
            pl.BlockSpec((D, H), lambda i: (0, 0)),
            pl.BlockSpec((H,), lambda i: (0,)),
        ],
        out_specs=pl.BlockSpec((bl, H), lambda i: (i, 0)),
        out_shape=jax.ShapeDtypeStruct((N, H), jnp.float32),
    )(feats, gamma, beta, W1, b1)
    sqrt_half = np.sqrt(0.5).astype(np.float32)
    h = 0.5 * preact * jax.lax.erfc(-preact * sqrt_half)
    scores = pl.pallas_call(
        _score2_body,
        grid=(N // bl,),
        in_specs=[
            pl.BlockSpec((bl, H), lambda i: (i, 0)),
            pl.BlockSpec((H, 1), lambda i: (0, 0)),
            pl.BlockSpec((1,), lambda i: (0,)),
        ],
        out_specs=pl.BlockSpec((bl // 128, 128), lambda i: (i, 0)),
        out_shape=jax.ShapeDtypeStruct((N // 128, 128), jnp.float32),
    )(h, W2, b2)
    return scores.reshape(B, L)


_IMIN_PY = -(2**31)


def _topk_body(s_ref, idx_ref, kbuf, ibuf, iebuf, rankbuf, kref, pgbuf, pebuf):
    NC = s_ref.shape[1]
    K = K_SEL
    NS = K // 128

    sbits = jax.lax.bitcast_convert_type(s_ref[0], jnp.int32)
    key = jnp.where(sbits < 0, sbits ^ jnp.int32(0x7FFFFFFF), sbits)

    def bit_step(i, t):
        cand = t | jax.lax.shift_left(jnp.int32(1), jnp.int32(30) - i)
        cnt = jnp.sum((key >= cand).astype(jnp.int32))
        return jnp.where(cnt >= K, cand, t)

    cnt0 = jnp.sum((key >= 0).astype(jnp.int32))
    t0 = jnp.where(cnt0 >= K, jnp.int32(0), jnp.int32(_IMIN_PY))
    tstar = jax.lax.fori_loop(0, 31, bit_step, t0)
    c_gt = jnp.sum((key > tstar).astype(jnp.int32))

    mask_gt = key > tstar
    mask_eq = key == tstar
    kref[...] = key

    tri_l = (jax.lax.broadcasted_iota(jnp.int32, (128, 128), 0)
             < jax.lax.broadcasted_iota(jnp.int32, (128, 128), 1)
             ).astype(jnp.float32)
    pgbuf[...] = jax.lax.dot_general(mask_gt.astype(jnp.float32), tri_l,
                                     (((1,), (0,)), ((), ()))).astype(jnp.int32)
    pebuf[...] = jax.lax.dot_general(mask_eq.astype(jnp.float32), tri_l,
                                     (((1,), (0,)), ((), ()))).astype(jnp.int32)

    kbuf[...] = jnp.full((NS + 2, 1, 128), jnp.int32(_IMIN_PY), jnp.int32)
    ibuf[...] = jnp.zeros((NS + 2, 1, 128), jnp.int32)
    iebuf[...] = jnp.zeros((NS + 2, 1, 128), jnp.int32)

    wio256 = jax.lax.broadcasted_iota(jnp.int32, (128, 256), 1)
    lane1 = jax.lax.broadcasted_iota(jnp.int32, (1, 128), 1)

    def compact_step(c, carry):
        wk, wi, we, rg, shg, re, she = carry
        kc = kref[pl.ds(c, 1), :]
        pg = pgbuf[pl.ds(c, 1), :]
        pe = pebuf[pl.ds(c, 1), :]
        mg = kc > tstar
        me = kc == tstar
        gc = c * 128 + lane1
        cg = jnp.sum(mg.astype(jnp.int32))
        ce = jnp.sum(me.astype(jnp.int32))

        oh_g = ((pg.T + shg) == wio256) & mg.T
        oh_e = ((pe.T + she) == wio256) & me.T
        win_ig = jnp.sum(jnp.where(oh_g, gc.T, 0), axis=0, keepdims=True)
        win_kg = jnp.sum(jnp.where(oh_g, kc.T, 0), axis=0, keepdims=True)
        win_mg = jnp.sum(oh_g.astype(jnp.int32), axis=0, keepdims=True)
        win_ie = jnp.sum(jnp.where(oh_e, gc.T, 0), axis=0, keepdims=True)
        win_me = jnp.sum(oh_e.astype(jnp.int32), axis=0, keepdims=True)

        wk = jnp.where(win_mg > 0, win_kg, wk)
        wi = jnp.where(win_mg > 0, win_ig, wi)
        we = jnp.where(win_me > 0, win_ie, we)

        sg2 = shg + cg
        se2 = she + ce
        fg = sg2 >= 128
        fe = se2 >= 128

        @pl.when(fg)
        def _():
            kbuf[pl.ds(rg, 1), 0:1, :] = wk[:, 0:128].reshape(1, 1, 128)
            ibuf[pl.ds(rg, 1), 0:1, :] = wi[:, 0:128].reshape(1, 1, 128)

        @pl.when(fe)
        def _():
            iebuf[pl.ds(jnp.minimum(re, NS + 1), 1), 0:1, :] = \
                we[:, 0:128].reshape(1, 1, 128)

        imin_h = jnp.full((1, 128), jnp.int32(_IMIN_PY), jnp.int32)
        zero_h = jnp.zeros((1, 128), jnp.int32)
        wk = jnp.where(fg, jnp.concatenate([wk[:, 128:256], imin_h], axis=1), wk)
        wi = jnp.where(fg, jnp.concatenate([wi[:, 128:256], zero_h], axis=1), wi)
        we = jnp.where(fe, jnp.concatenate([we[:, 128:256], zero_h], axis=1), we)
        rg = rg + fg.astype(jnp.int32)
        re = re + fe.astype(jnp.int32)
        shg = sg2 - 128 * fg.astype(jnp.int32)
        she = se2 - 128 * fe.astype(jnp.int32)
        return (wk, wi, we, rg, shg, re, she)

    init = (jnp.full((1, 256), jnp.int32(_IMIN_PY), jnp.int32),
            jnp.zeros((1, 256), jnp.int32),
            jnp.zeros((1, 256), jnp.int32),
            jnp.int32(0), jnp.int32(0), jnp.int32(0), jnp.int32(0))
    wk, wi, we, rg, shg, re, she = jax.lax.fori_loop(0, NC, compact_step, init)
    kbuf[pl.ds(rg, 1), 0:1, :] = wk[:, 0:128].reshape(1, 1, 128)
    ibuf[pl.ds(rg, 1), 0:1, :] = wi[:, 0:128].reshape(1, 1, 128)
    iebuf[pl.ds(jnp.minimum(re, NS + 1), 1), 0:1, :] = \
        we[:, 0:128].reshape(1, 1, 128)

    def rank_strip(s, carry):
        ke = kbuf[pl.ds(s, 1), 0:1, :].reshape(1, 128)
        ie = ibuf[pl.ds(s, 1), 0:1, :].reshape(1, 128)

        def racc(fs, a):
            kfv = kbuf[pl.ds(fs, 1), 0:1, :].reshape(1, 128)
            ifv = ibuf[pl.ds(fs, 1), 0:1, :].reshape(1, 128)
            gtc = (kfv.T > ke).astype(jnp.int32)
            tie = ((kfv.T == ke) & (ifv.T < ie)).astype(jnp.int32)
            return a + jnp.sum(gtc + tie, axis=0, keepdims=True)

        r = jax.lax.fori_loop(0, NS, racc, jnp.zeros((1, 128), jnp.int32))
        r = jnp.where(ke == jnp.int32(_IMIN_PY), jnp.int32(2 * K), r)
        rankbuf[pl.ds(s, 1), :] = r
        return carry

    jax.lax.fori_loop(0, NS, rank_strip, 0)

    def out_strip(s, carry):
        pio = jax.lax.broadcasted_iota(jnp.int32, (1, 128), 1) + s * 128

        def inner(fs, a):
            rg = rankbuf[pl.ds(fs, 1), :]
            vg = ibuf[pl.ds(fs, 1), 0:1, :].reshape(1, 128).astype(jnp.float32)
            epos_row = fs * 128 + lane1
            re = jnp.where(epos_row < K - c_gt, epos_row + c_gt,
                           jnp.int32(2 * K))
            ve = iebuf[pl.ds(fs, 1), 0:1, :].reshape(1, 128).astype(jnp.float32)
            ohg = (rg.T == pio).astype(jnp.float32)
            ohe = (re.T == pio).astype(jnp.float32)
            hp = jax.lax.Precision.HIGHEST
            a = a + jax.lax.dot_general(vg, ohg, (((1,), (0,)), ((), ())),
                                        precision=hp)
            a = a + jax.lax.dot_general(ve, ohe, (((1,), (0,)), ((), ())),
                                        precision=hp)
            return a

        acc = jax.lax.fori_loop(0, NS, inner, jnp.zeros((1, 128), jnp.float32))
        idx_ref[0, pl.ds(s, 1), :] = acc.astype(jnp.int32)
        return carry

    jax.lax.fori_loop(0, NS, out_strip, 0)


def _topk(scores):
    B, L = scores.shape
    K = K_SEL
    s3 = scores.reshape(B, L // 128, 128)
    idx = pl.pallas_call(
        _topk_body,
        grid=(B,),
        in_specs=[pl.BlockSpec((1, L // 128, 128), lambda b: (b, 0, 0))],
        out_specs=pl.BlockSpec((1, K // 128, 128), lambda b: (b, 0, 0)),
        out_shape=jax.ShapeDtypeStruct((B, K // 128, 128), jnp.int32),
        scratch_shapes=[
            pltpu.VMEM((K // 128 + 2, 1, 128), jnp.int32),
            pltpu.VMEM((K // 128 + 2, 1, 128), jnp.int32),
            pltpu.VMEM((K // 128 + 2, 1, 128), jnp.int32),
            pltpu.VMEM((K // 128, 128), jnp.int32),
            pltpu.VMEM((L // 128, 128), jnp.int32),
            pltpu.VMEM((L // 128, 128), jnp.int32),
            pltpu.VMEM((L // 128, 128), jnp.int32),
        ],
    )(s3)
    return idx.reshape(B, K)


def kernel(features, k, gamma, beta, W1, b1, W2, b2):
    scores = _scores(features, gamma, beta, W1, b1, W2, b2)
    idx = _topk(scores)
    idx = idx + (jnp.asarray(k, dtype=idx.dtype) - K_SEL)
    selected = jnp.take_along_axis(features, idx[:, :, None], axis=1)
    return selected, scores, idx
